# baseline (speedup 1.0000x reference)
import jax
import jax.numpy as jnp
from jax.experimental import pallas as pl
from jax.experimental.pallas import tpu as pltpu

_NBUF = 6
_CH = 4


def _make_body(B, H, W, C):
    HH = H // _CH
    NCHUNK = B * _CH

    def body(x_hbm, pos_hbm, o_hbm, xbuf, obuf, pos_v, insem, outsem, possem):
        pos_cp = pltpu.make_async_copy(
            pos_hbm.at[pl.ds(0, H), pl.ds(0, W), :], pos_v, possem
        )
        pos_cp.start()

        def in_cp(k, slot):
            b, half = divmod(k, _CH)
            return pltpu.make_async_copy(
                x_hbm.at[b, pl.ds(half * HH, HH)], xbuf.at[slot],
                insem.at[slot],
            )

        def out_cp(k, slot):
            b, half = divmod(k, _CH)
            return pltpu.make_async_copy(
                obuf.at[slot], o_hbm.at[b, pl.ds(half * HH, HH)],
                outsem.at[slot],
            )

        for k in range(_NBUF):
            in_cp(k, k).start()
        pos_cp.wait()
        for k in range(NCHUNK):
            slot = k % _NBUF
            half = k % _CH
            in_cp(k, slot).wait()
            if k >= _NBUF:
                out_cp(k - _NBUF, slot).wait()
            obuf[slot] = xbuf[slot] + pos_v[pl.ds(half * HH, HH)]
            if k + _NBUF < NCHUNK:
                in_cp(k + _NBUF, slot).start()
            out_cp(k, slot).start()
        for k in range(NCHUNK - _NBUF, NCHUNK):
            out_cp(k, k % _NBUF).wait()

    return body


def kernel(x, pos_embed):
    B, H, W, C = x.shape
    HH = H // _CH
    out = pl.pallas_call(
        _make_body(B, H, W, C),
        in_specs=[
            pl.BlockSpec(memory_space=pltpu.MemorySpace.HBM),
            pl.BlockSpec(memory_space=pltpu.MemorySpace.HBM),
        ],
        out_specs=pl.BlockSpec(memory_space=pltpu.MemorySpace.HBM),
        out_shape=jax.ShapeDtypeStruct(x.shape, x.dtype),
        scratch_shapes=[
            pltpu.VMEM((_NBUF, HH, W, C), jnp.float32),
            pltpu.VMEM((_NBUF, HH, W, C), jnp.float32),
            pltpu.VMEM((H, W, C), jnp.float32),
            pltpu.SemaphoreType.DMA((_NBUF,)),
            pltpu.SemaphoreType.DMA((_NBUF,)),
            pltpu.SemaphoreType.DMA,
        ],
    )(x, pos_embed)
    return out


# speedup vs baseline: 1.0148x; 1.0148x over previous
import jax
import jax.numpy as jnp
from jax.experimental import pallas as pl
from jax.experimental.pallas import tpu as pltpu

_NBUF = 6
_CH = 2


def _make_body(B, H, W, C):
    HH = H // _CH
    NCHUNK = B * _CH

    def body(x_hbm, pos_hbm, o_hbm, xbuf, obuf, pos_v, insem, outsem, possem):
        pos_cp = pltpu.make_async_copy(
            pos_hbm.at[pl.ds(0, H), pl.ds(0, W), :], pos_v, possem
        )
        pos_cp.start()

        def in_cp(k, slot):
            b, half = divmod(k, _CH)
            return pltpu.make_async_copy(
                x_hbm.at[b, pl.ds(half * HH, HH)], xbuf.at[slot],
                insem.at[slot],
            )

        def out_cp(k, slot):
            b, half = divmod(k, _CH)
            return pltpu.make_async_copy(
                obuf.at[slot], o_hbm.at[b, pl.ds(half * HH, HH)],
                outsem.at[slot],
            )

        for k in range(_NBUF):
            in_cp(k, k).start()
        pos_cp.wait()
        for k in range(NCHUNK):
            slot = k % _NBUF
            half = k % _CH
            in_cp(k, slot).wait()
            if k >= _NBUF:
                out_cp(k - _NBUF, slot).wait()
            obuf[slot] = xbuf[slot] + pos_v[pl.ds(half * HH, HH)]
            if k + _NBUF < NCHUNK:
                in_cp(k + _NBUF, slot).start()
            out_cp(k, slot).start()
        for k in range(NCHUNK - _NBUF, NCHUNK):
            out_cp(k, k % _NBUF).wait()

    return body


def kernel(x, pos_embed):
    B, H, W, C = x.shape
    HH = H // _CH
    out = pl.pallas_call(
        _make_body(B, H, W, C),
        in_specs=[
            pl.BlockSpec(memory_space=pltpu.MemorySpace.HBM),
            pl.BlockSpec(memory_space=pltpu.MemorySpace.HBM),
        ],
        out_specs=pl.BlockSpec(memory_space=pltpu.MemorySpace.HBM),
        out_shape=jax.ShapeDtypeStruct(x.shape, x.dtype),
        scratch_shapes=[
            pltpu.VMEM((_NBUF, HH, W, C), jnp.float32),
            pltpu.VMEM((_NBUF, HH, W, C), jnp.float32),
            pltpu.VMEM((H, W, C), jnp.float32),
            pltpu.SemaphoreType.DMA((_NBUF,)),
            pltpu.SemaphoreType.DMA((_NBUF,)),
            pltpu.SemaphoreType.DMA,
        ],
    )(x, pos_embed)
    return out


# speedup vs baseline: 1.0185x; 1.0037x over previous
import jax
import jax.numpy as jnp
from jax.experimental import pallas as pl
from jax.experimental.pallas import tpu as pltpu

_NBUF = 12
_OUT_SLACK = 6
_CH = 2


def _make_body(B, H, W, C):
    HH = H // _CH
    NCHUNK = B * _CH

    def body(x_hbm, pos_hbm, o_hbm, buf, pos_v, insem, outsem, possem):
        pos_cp = pltpu.make_async_copy(
            pos_hbm.at[pl.ds(0, H), pl.ds(0, W), :], pos_v, possem
        )
        pos_cp.start()

        def in_cp(k):
            b, half = divmod(k, _CH)
            slot = k % _NBUF
            return pltpu.make_async_copy(
                x_hbm.at[b, pl.ds(half * HH, HH)], buf.at[slot],
                insem.at[slot],
            )

        def out_cp(k):
            b, half = divmod(k, _CH)
            slot = k % _NBUF
            return pltpu.make_async_copy(
                buf.at[slot], o_hbm.at[b, pl.ds(half * HH, HH)],
                outsem.at[slot],
            )

        for k in range(_NBUF):
            in_cp(k).start()
        pos_cp.wait()
        for k in range(NCHUNK):
            slot = k % _NBUF
            half = k % _CH
            in_cp(k).wait()
            buf[slot] = buf[slot] + pos_v[pl.ds(half * HH, HH)]
            out_cp(k).start()
            done = k - _OUT_SLACK
            if done >= 0:
                out_cp(done).wait()
                if done + _NBUF < NCHUNK:
                    in_cp(done + _NBUF).start()
        for k in range(NCHUNK - _OUT_SLACK, NCHUNK):
            out_cp(k).wait()

    return body


def kernel(x, pos_embed):
    B, H, W, C = x.shape
    HH = H // _CH
    out = pl.pallas_call(
        _make_body(B, H, W, C),
        in_specs=[
            pl.BlockSpec(memory_space=pltpu.MemorySpace.HBM),
            pl.BlockSpec(memory_space=pltpu.MemorySpace.HBM),
        ],
        out_specs=pl.BlockSpec(memory_space=pltpu.MemorySpace.HBM),
        out_shape=jax.ShapeDtypeStruct(x.shape, x.dtype),
        scratch_shapes=[
            pltpu.VMEM((_NBUF, HH, W, C), jnp.float32),
            pltpu.VMEM((H, W, C), jnp.float32),
            pltpu.SemaphoreType.DMA((_NBUF,)),
            pltpu.SemaphoreType.DMA((_NBUF,)),
            pltpu.SemaphoreType.DMA,
        ],
    )(x, pos_embed)
    return out
